# baseline (speedup 1.0000x reference)
import functools

import jax
import jax.numpy as jnp
from jax import lax
from jax.experimental import pallas as pl
from jax.experimental.pallas import tpu as pltpu
from jax.experimental.pallas import tpu_sc as plsc

NC = 2
NS = 16
NW = NC * NS
L = 16

N = 10000
D = 128
NPAD = 10240
R = NPAD // NW
TRASH = R

C = 1280
G = 64

NEG = float("-inf")


def _sc_body(x_hbm, src_hbm, dst_hbm, m_hbm,
             acc, src_c, dst_c, pend_src, pend_ldst, rows, sem):
    cid = lax.axis_index("c")
    sid = lax.axis_index("s")
    wid = sid * NC + cid
    lo = wid * R

    neg_vec = jnp.full((L,), NEG, jnp.float32)

    def init_body(i, _):
        acc[pl.ds(i * L, L)] = neg_vec
        return 0

    lax.fori_loop(0, (R + 1) * D // L, init_body, 0)

    n_chunks = src_hbm.shape[0] // C

    def chunk_body(ci, _):
        base = ci * C
        pltpu.sync_copy(src_hbm.at[pl.ds(base, C)], src_c)
        pltpu.sync_copy(dst_hbm.at[pl.ds(base, C)], dst_c)

        def scan_body(i, ptr):
            s = src_c[pl.ds(i * L, L)]
            d = dst_c[pl.ds(i * L, L)]
            ld = d - lo
            mask = (ld >= 0) & (ld < R)
            key = (1 - mask.astype(jnp.int32)).astype(jnp.uint32)
            val = s * 512 + ld
            _, vs = plsc.sort_key_val(key, val)
            pend_src[pl.ds(ptr, L)] = lax.shift_right_logical(vs, 9)
            pend_ldst[pl.ds(ptr, L)] = vs & 511
            cnt = plsc.all_reduce_population_count(mask)[0]
            return ptr + cnt

        k = lax.fori_loop(0, C // L, scan_body, 0)

        zero_vec = jnp.zeros((L,), jnp.int32)
        trash_vec = jnp.full((L,), TRASH, jnp.int32)
        for j in range(G // L):
            pend_src[pl.ds(k + j * L, L)] = zero_vec
            pend_ldst[pl.ds(k + j * L, L)] = trash_vec

        ng = (k + G - 1) // G

        def batch_body(g, _):
            idx = pend_src.at[pl.ds(g * G, G)]
            pltpu.async_copy(x_hbm.at[idx], rows, sem).wait()
            for gi in range(0):
                lv = pend_ldst[pl.ds(g * G + gi * L, L)]
                for j in range(L):
                    rb = lv[j] * D
                    for f in range(D // L):
                        a = acc[pl.ds(rb + f * L, L)]
                        v = rows[gi * L + j, pl.ds(f * L, L)]
                        acc[pl.ds(rb + f * L, L)] = jnp.maximum(a, v)
            return 0

        lax.fori_loop(0, ng, batch_body, 0)
        return 0

    lax.fori_loop(0, n_chunks, chunk_body, 0)

    pltpu.sync_copy(acc.at[pl.ds(0, R * D)], m_hbm.at[pl.ds(lo * D, R * D)])


def _sc_segmax(x, src, dst):
    mesh = plsc.VectorSubcoreMesh(core_axis_name="c", subcore_axis_name="s")
    f = pl.kernel(
        _sc_body,
        out_type=jax.ShapeDtypeStruct((NPAD * D,), jnp.float32),
        mesh=mesh,
        scratch_types=[
            pltpu.VMEM(((R + 1) * D,), jnp.float32),
            pltpu.VMEM((C,), jnp.int32),
            pltpu.VMEM((C,), jnp.int32),
            pltpu.VMEM((C + G,), jnp.int32),
            pltpu.VMEM((C + G,), jnp.int32),
            pltpu.VMEM((G, D), jnp.float32),
            pltpu.SemaphoreType.DMA,
        ],
        compiler_params=pltpu.CompilerParams(needs_layout_passes=False),
    )
    return f(x, src, dst)


BLK = 1024


def _tc_body(x_ref, m_ref, w_ref, b_ref, o_ref):
    xb = x_ref[...]
    mb = m_ref[...]
    md = jnp.where(mb > NEG, mb - xb, jnp.float32(0.0))
    w1 = w_ref[:, :D]
    w2 = w_ref[:, D:]
    dims = (((1,), (1,)), ((), ()))
    o_ref[...] = (
        lax.dot_general(xb, w1, dims, preferred_element_type=jnp.float32)
        + lax.dot_general(md, w2, dims, preferred_element_type=jnp.float32)
        + b_ref[...]
    )


def _tc_matmul(xp, m2d, W, b):
    grid = (NPAD // BLK,)
    return pl.pallas_call(
        _tc_body,
        grid=grid,
        in_specs=[
            pl.BlockSpec((BLK, D), lambda i: (i, 0)),
            pl.BlockSpec((BLK, D), lambda i: (i, 0)),
            pl.BlockSpec((D, 2 * D), lambda i: (0, 0)),
            pl.BlockSpec((1, D), lambda i: (0, 0)),
        ],
        out_specs=pl.BlockSpec((BLK, D), lambda i: (i, 0)),
        out_shape=jax.ShapeDtypeStruct((NPAD, D), jnp.float32),
    )(xp, m2d, W, b)


def kernel(x, edge_index, W, b):
    src = edge_index[0]
    dst = edge_index[1]
    m_flat = _sc_segmax(x, src, dst)
    m2d = m_flat.reshape(NPAD, D)
    xp = jnp.pad(x, ((0, NPAD - N), (0, 0)))
    out = _tc_matmul(xp, m2d, W, b.reshape(1, D))
    return out[:N]


# speedup vs baseline: 10.1743x; 10.1743x over previous
import functools

import jax
import jax.numpy as jnp
from jax import lax
from jax.experimental import pallas as pl
from jax.experimental.pallas import tpu as pltpu
from jax.experimental.pallas import tpu_sc as plsc

NC = 2
NS = 16
NW = NC * NS
L = 16

N = 10000
D = 128
NPAD = 10240
F = D // NW
TRASH = F * NPAD

C = 8000

NEG = float("-inf")


def _sc_body(xt_hbm, src_hbm, dst_hbm, mt_hbm, acc, xs, src_c0, src_c1,
             dst_c0, dst_c1, sem_x, sem_s, sem_d):
    src_c = (src_c0, src_c1)
    dst_c = (dst_c0, dst_c1)
    cid = lax.axis_index("c")
    sid = lax.axis_index("s")
    wid = sid * NC + cid

    E = src_hbm.shape[0]
    n_chunks = E // C

    xcp = pltpu.async_copy(
        xt_hbm.at[pl.ds(wid * F * NPAD, F * NPAD)], xs, sem_x)

    def chunk_off(ci):
        f = ci + wid * (n_chunks // NW)
        return jnp.where(f >= n_chunks, f - n_chunks, f) * C

    b0 = chunk_off(0)
    pltpu.async_copy(src_hbm.at[pl.ds(b0, C)], src_c[0], sem_s)
    pltpu.async_copy(dst_hbm.at[pl.ds(b0, C)], dst_c[0], sem_d)

    neg_vec = jnp.full((L,), NEG, jnp.float32)

    def init_body(i, _):
        acc[pl.ds(i * L, L)] = neg_vec
        return 0

    lax.fori_loop(0, (F * NPAD + L) // L, init_body, 0)
    xcp.wait()

    lane = lax.iota(jnp.int32, L)
    seg_steps = []
    for k in (1, 2, 4, 8):
        idx_up = jnp.maximum(lane - k, 0)
        ge_k = lane >= k
        seg_steps.append((idx_up, ge_k))
    idx_dn = jnp.minimum(lane + 1, L - 1)
    lt_last = lane < (L - 1)
    trash_vec = jnp.full((L,), TRASH, jnp.int32)

    def process_group(g, slot):
        s = src_c[slot][pl.ds(g * L, L)]
        d = dst_c[slot][pl.ds(g * L, L)]
        ks, ss = plsc.sort_key_val(d, s)
        segm = [(ks.at[iu].get(mode="promise_in_bounds") == ks) & gk
                for iu, gk in seg_steps]
        nxt = ks.at[idx_dn].get(mode="promise_in_bounds")
        not_last = (nxt == ks) & lt_last
        avs = [plsc.load_gather(acc, [ks + f * NPAD]) for f in range(F)]
        mvs = []
        for f in range(F):
            xv = plsc.load_gather(xs, [ss + f * NPAD])
            for (iu, _), m in zip(seg_steps, segm):
                sh = xv.at[iu].get(mode="promise_in_bounds")
                xv = jnp.where(m, jnp.maximum(xv, sh), xv)
            mvs.append(jnp.maximum(xv, avs[f]))
        for f in range(F):
            posf = jnp.where(not_last, trash_vec + f, ks + f * NPAD)
            plsc.store_scatter(acc, [posf], mvs[f])

    def process_chunk(ci, slot):
        @pl.when(ci + 1 < n_chunks)
        def _():
            nb = chunk_off(ci + 1)
            pltpu.async_copy(src_hbm.at[pl.ds(nb, C)], src_c[1 - slot],
                             sem_s)
            pltpu.async_copy(dst_hbm.at[pl.ds(nb, C)], dst_c[1 - slot],
                             sem_d)

        cb = chunk_off(ci)
        pltpu.make_async_copy(src_hbm.at[pl.ds(cb, C)], src_c[slot],
                              sem_s).wait()
        pltpu.make_async_copy(dst_hbm.at[pl.ds(cb, C)], dst_c[slot],
                              sem_d).wait()

        def scan_body(i, _):
            process_group(i * 2, slot)
            process_group(i * 2 + 1, slot)
            return 0

        lax.fori_loop(0, C // L // 2, scan_body, 0)

    def chunk_pair(o, _):
        process_chunk(o * 2, 0)
        process_chunk(o * 2 + 1, 1)
        return 0

    lax.fori_loop(0, n_chunks // 2, chunk_pair, 0)

    pltpu.sync_copy(acc.at[pl.ds(0, F * NPAD)],
                    mt_hbm.at[pl.ds(wid * F * NPAD, F * NPAD)])


def _sc_segmax(xt_flat, src, dst):
    mesh = plsc.VectorSubcoreMesh(core_axis_name="c", subcore_axis_name="s")
    f = pl.kernel(
        _sc_body,
        out_type=jax.ShapeDtypeStruct((D * NPAD,), jnp.float32),
        mesh=mesh,
        scratch_types=[
            pltpu.VMEM((F * NPAD + L,), jnp.float32),
            pltpu.VMEM((F * NPAD,), jnp.float32),
            pltpu.VMEM((C,), jnp.int32),
            pltpu.VMEM((C,), jnp.int32),
            pltpu.VMEM((C,), jnp.int32),
            pltpu.VMEM((C,), jnp.int32),
            pltpu.SemaphoreType.DMA,
            pltpu.SemaphoreType.DMA,
            pltpu.SemaphoreType.DMA,
        ],
        compiler_params=pltpu.CompilerParams(needs_layout_passes=False),
    )
    return f(xt_flat, src, dst)


BLK = 1024


def _tc_body(x_ref, xt_ref, mt_ref, w_ref, b_ref, o_ref):
    xb = x_ref[...]
    xtb = xt_ref[...]
    mtb = mt_ref[...]
    mdt = jnp.where(mtb > NEG, mtb - xtb, jnp.float32(0.0))
    w1 = w_ref[:, :D]
    w2 = w_ref[:, D:]
    o_ref[...] = (
        lax.dot_general(xb, w1, (((1,), (1,)), ((), ())),
                        preferred_element_type=jnp.float32)
        + lax.dot_general(mdt, w2, (((0,), (1,)), ((), ())),
                          preferred_element_type=jnp.float32)
        + b_ref[...]
    )


def _tc_matmul(xp, xtp, mt, W, b):
    grid = (NPAD // BLK,)
    return pl.pallas_call(
        _tc_body,
        grid=grid,
        in_specs=[
            pl.BlockSpec((BLK, D), lambda i: (i, 0)),
            pl.BlockSpec((D, BLK), lambda i: (0, i)),
            pl.BlockSpec((D, BLK), lambda i: (0, i)),
            pl.BlockSpec((D, 2 * D), lambda i: (0, 0)),
            pl.BlockSpec((1, D), lambda i: (0, 0)),
        ],
        out_specs=pl.BlockSpec((BLK, D), lambda i: (i, 0)),
        out_shape=jax.ShapeDtypeStruct((NPAD, D), jnp.float32),
    )(xp, xtp, mt, W, b)


def kernel(x, edge_index, W, b):
    src = edge_index[0]
    dst = edge_index[1]
    xtp = jnp.pad(x.T, ((0, 0), (0, NPAD - N)))
    mt_flat = _sc_segmax(xtp.reshape(-1), src, dst)
    mt = mt_flat.reshape(D, NPAD)
    xp = jnp.pad(x, ((0, NPAD - N), (0, 0)))
    out = _tc_matmul(xp, xtp, mt, W, b.reshape(1, D))
    return out[:N]


# speedup vs baseline: 16.4364x; 1.6155x over previous
import functools

import jax
import jax.numpy as jnp
from jax import lax
from jax.experimental import pallas as pl
from jax.experimental.pallas import tpu as pltpu
from jax.experimental.pallas import tpu_sc as plsc

NC = 2
NS = 16
NW = NC * NS
L = 16

N = 10000
D = 128
NPAD = 10240
F = D // NW
TRASH = F * NPAD

C = 8000

NEG = float("-inf")


def _sc_body(xt_hbm, src_hbm, dst_hbm, mt_hbm, acc, xs, src_c0, src_c1,
             dst_c0, dst_c1, sem_x, sem_s, sem_d):
    src_c = (src_c0, src_c1)
    dst_c = (dst_c0, dst_c1)
    cid = lax.axis_index("c")
    sid = lax.axis_index("s")
    wid = sid * NC + cid

    E = src_hbm.shape[0]
    n_chunks = E // C

    xcp = pltpu.async_copy(
        xt_hbm.at[pl.ds(wid * F * NPAD, F * NPAD)], xs, sem_x)

    def chunk_off(ci):
        f = ci + wid * (n_chunks // NW)
        return jnp.where(f >= n_chunks, f - n_chunks, f) * C

    b0 = chunk_off(0)
    pltpu.async_copy(src_hbm.at[pl.ds(b0, C)], src_c[0], sem_s)
    pltpu.async_copy(dst_hbm.at[pl.ds(b0, C)], dst_c[0], sem_d)

    neg_vec = jnp.full((L,), NEG, jnp.float32)

    def init_body(i, _):
        acc[pl.ds(i * L, L)] = neg_vec
        return 0

    lax.fori_loop(0, (F * NPAD + L) // L, init_body, 0)
    xcp.wait()

    lane = lax.iota(jnp.int32, L)
    seg_idx = [jnp.maximum(lane - k, 0) for k in (1, 2, 4, 8)]
    idx_dn = jnp.minimum(lane + 1, L - 1)
    lt_last = lane < (L - 1)
    trash_vec = jnp.full((L,), N, jnp.int32)

    def sort_group(g, slot):
        s = src_c[slot][pl.ds(g * L, L)]
        d = dst_c[slot][pl.ds(g * L, L)]
        ksu, ss = plsc.sort_key_val(plsc.bitcast(d, jnp.uint32), s)
        return plsc.bitcast(ksu, jnp.int32), ss

    def process_group(ks, ss):
        segi = [jnp.where(ks.at[iu].get(mode="promise_in_bounds") == ks,
                          iu, lane)
                for iu in seg_idx]
        nxt = ks.at[idx_dn].get(mode="promise_in_bounds")
        not_last = (nxt == ks) & lt_last
        pos0 = jnp.where(not_last, trash_vec, ks)
        avs = [plsc.load_gather(acc.at[pl.ds(f * NPAD, NPAD)], [ks])
               for f in range(F)]
        mvs = []
        for f in range(F):
            xv = plsc.load_gather(xs.at[pl.ds(f * NPAD, NPAD)], [ss])
            for si in segi:
                xv = jnp.maximum(xv, xv.at[si].get(mode="promise_in_bounds"))
            mvs.append(jnp.maximum(xv, avs[f]))
        for f in range(F):
            plsc.store_scatter(acc.at[pl.ds(f * NPAD, NPAD)], [pos0], mvs[f])

    def process_chunk(ci, slot):
        @pl.when(ci + 1 < n_chunks)
        def _():
            nb = chunk_off(ci + 1)
            pltpu.async_copy(src_hbm.at[pl.ds(nb, C)], src_c[1 - slot],
                             sem_s)
            pltpu.async_copy(dst_hbm.at[pl.ds(nb, C)], dst_c[1 - slot],
                             sem_d)

        cb = chunk_off(ci)
        pltpu.make_async_copy(src_hbm.at[pl.ds(cb, C)], src_c[slot],
                              sem_s).wait()
        pltpu.make_async_copy(dst_hbm.at[pl.ds(cb, C)], dst_c[slot],
                              sem_d).wait()

        def scan_body(i, _):
            g0 = i * 4
            kv0 = sort_group(g0, slot)
            kv1 = sort_group(g0 + 1, slot)
            process_group(*kv0)
            kv2 = sort_group(g0 + 2, slot)
            process_group(*kv1)
            kv3 = sort_group(g0 + 3, slot)
            process_group(*kv2)
            process_group(*kv3)
            return 0

        lax.fori_loop(0, C // L // 4, scan_body, 0)

    def chunk_pair(o, _):
        process_chunk(o * 2, 0)
        process_chunk(o * 2 + 1, 1)
        return 0

    lax.fori_loop(0, n_chunks // 2, chunk_pair, 0)

    pltpu.sync_copy(acc.at[pl.ds(0, F * NPAD)],
                    mt_hbm.at[pl.ds(wid * F * NPAD, F * NPAD)])


def _sc_segmax(xt_flat, src, dst):
    mesh = plsc.VectorSubcoreMesh(core_axis_name="c", subcore_axis_name="s")
    f = pl.kernel(
        _sc_body,
        out_type=jax.ShapeDtypeStruct((D * NPAD,), jnp.float32),
        mesh=mesh,
        scratch_types=[
            pltpu.VMEM((F * NPAD + L,), jnp.float32),
            pltpu.VMEM((F * NPAD,), jnp.float32),
            pltpu.VMEM((C,), jnp.int32),
            pltpu.VMEM((C,), jnp.int32),
            pltpu.VMEM((C,), jnp.int32),
            pltpu.VMEM((C,), jnp.int32),
            pltpu.SemaphoreType.DMA,
            pltpu.SemaphoreType.DMA,
            pltpu.SemaphoreType.DMA,
        ],
        compiler_params=pltpu.CompilerParams(needs_layout_passes=False),
    )
    return f(xt_flat, src, dst)


BLK = 1024


def _tc_body(x_ref, xt_ref, mt_ref, w_ref, b_ref, o_ref):
    xb = x_ref[...]
    xtb = xt_ref[...]
    mtb = mt_ref[...]
    mdt = jnp.where(mtb > NEG, mtb - xtb, jnp.float32(0.0))
    w1 = w_ref[:, :D]
    w2 = w_ref[:, D:]
    o_ref[...] = (
        lax.dot_general(xb, w1, (((1,), (1,)), ((), ())),
                        preferred_element_type=jnp.float32)
        + lax.dot_general(mdt, w2, (((0,), (1,)), ((), ())),
                          preferred_element_type=jnp.float32)
        + b_ref[...]
    )


def _tc_matmul(xp, xtp, mt, W, b):
    grid = (NPAD // BLK,)
    return pl.pallas_call(
        _tc_body,
        grid=grid,
        in_specs=[
            pl.BlockSpec((BLK, D), lambda i: (i, 0)),
            pl.BlockSpec((D, BLK), lambda i: (0, i)),
            pl.BlockSpec((D, BLK), lambda i: (0, i)),
            pl.BlockSpec((D, 2 * D), lambda i: (0, 0)),
            pl.BlockSpec((1, D), lambda i: (0, 0)),
        ],
        out_specs=pl.BlockSpec((BLK, D), lambda i: (i, 0)),
        out_shape=jax.ShapeDtypeStruct((NPAD, D), jnp.float32),
    )(xp, xtp, mt, W, b)


def kernel(x, edge_index, W, b):
    src = edge_index[0]
    dst = edge_index[1]
    xtp = jnp.pad(x.T, ((0, 0), (0, NPAD - N)))
    mt_flat = _sc_segmax(xtp.reshape(-1), src, dst)
    mt = mt_flat.reshape(D, NPAD)
    xp = jnp.pad(x, ((0, NPAD - N), (0, 0)))
    out = _tc_matmul(xp, xtp, mt, W, b.reshape(1, D))
    return out[:N]
